# baseline (speedup 1.0000x reference)
import functools

import jax
import jax.numpy as jnp
from jax import lax
from jax.experimental import pallas as pl
from jax.experimental.pallas import tpu as pltpu
from jax.experimental.pallas import tpu_sc as plsc

NC = 2
NS = 16
L = 16

N = 10000
E = 160000
D = 256
H = 128
NAG = 1024

CH = 80
NCHUNK = 128
EPAD = NS * NCHUNK * CH
NP = N + 8
GB = 8
G = NCHUNK // GB
NRING = 4
OUTER = 4
NBUF = 4
WCH = 40
NWCH = N // WCH
WPT = (NWCH + NS - 1) // NS
AGT = NAG // NS


def _sc_mesh():
    return plsc.VectorSubcoreMesh(core_axis_name="c", subcore_axis_name="s",
                                  num_cores=NC, num_subcores=NS)


@functools.cache
def _sc_segsum_call():
    return functools.partial(
        pl.kernel,
        out_type=[
            jax.ShapeDtypeStruct((N, H), jnp.float32),
            jax.ShapeDtypeStruct((N, H), jnp.float32),
        ],
        mesh=_sc_mesh(),
        scratch_types=[
            pltpu.VMEM((GB, CH), jnp.int32),
            pltpu.VMEM((GB, CH), jnp.int32),
            pltpu.VMEM((GB, CH), jnp.int32),
            pltpu.VMEM((GB, CH), jnp.int32),
            pltpu.VMEM((GB, CH), jnp.int32),
            pltpu.VMEM((GB, CH), jnp.int32),
            pltpu.VMEM((GB, CH), jnp.int32),
            pltpu.VMEM((GB, CH), jnp.int32),
            pltpu.VMEM((CH, H), jnp.float32),
            pltpu.VMEM((CH, H), jnp.float32),
            pltpu.VMEM((CH, H), jnp.float32),
            pltpu.VMEM((CH, H), jnp.float32),
            pltpu.VMEM_SHARED((NP, H), jnp.float32),
            pltpu.SemaphoreType.DMA,
            pltpu.SemaphoreType.DMA,
            pltpu.SemaphoreType.DMA,
            pltpu.SemaphoreType.DMA,
            pltpu.SemaphoreType.DMA,
            pltpu.SemaphoreType.DMA,
            pltpu.SemaphoreType.DMA,
        ],
    )(_sc_segsum_body)


def _sc_segsum_body(p_lo, p_hi, src2, dst2, out_lo, out_hi,
                    sr0, sr1, sr2, sr3, dr0, dr1, dr2, dr3,
                    gb0, gb1, gb2, gb3, acc,
                    gsem, ss0, ss1, ss2, ss3, is0, is1):
    srings = (sr0, sr1, sr2, sr3)
    drings = (dr0, dr1, dr2, dr3)
    bufs = (gb0, gb1, gb2, gb3)
    ssems = (ss0, ss1, ss2, ss3)
    isems = (is0, is1)
    c = lax.axis_index("c")
    s = lax.axis_index("s")

    def zrow(r, carry):
        for j in range(H // L):
            gb0[r, pl.ds(j * L, L)] = jnp.zeros((L,), jnp.float32)
        return carry

    lax.fori_loop(0, WCH, zrow, 0)
    zsrc = gb0.at[pl.ds(0, WCH)]
    for k in range(WPT):
        cidx = s + k * NS

        @pl.when(cidx < NWCH)
        def _():
            pltpu.sync_copy(zsrc, acc.at[pl.ds(cidx * WCH, WCH)])
    plsc.subcore_barrier()

    def stage(g, r, parity):
        base = pl.multiple_of(s * NCHUNK + g * GB, GB)
        pltpu.async_copy(src2.at[pl.ds(base, GB)], srings[r], isems[parity])
        pltpu.async_copy(dst2.at[pl.ds(base, GB)], drings[r], isems[parity])

    def stage_wait(parity):
        pltpu.make_async_copy(src2.at[pl.ds(0, GB)], srings[0],
                              isems[parity]).wait()
        pltpu.make_async_copy(dst2.at[pl.ds(0, GB)], drings[0],
                              isems[parity]).wait()

    def run(p_hbm):
        def startg(ring, row, b):
            pltpu.async_copy(p_hbm.at[srings[ring].at[row]], bufs[b], gsem)

        def waitg(b):
            pltpu.make_async_copy(p_hbm.at[srings[0].at[0]], bufs[b],
                                  gsem).wait()

        def starts(ring, row, b):
            pltpu.async_copy(bufs[b], acc.at[drings[ring].at[row]],
                             ssems[b], add=True)

        def waits(b):
            pltpu.make_async_copy(bufs[b], acc.at[drings[0].at[0]],
                                  ssems[b]).wait()

        stage(0, 0, 0)
        stage(1, 1, 1)
        stage_wait(0)
        for j in range(NBUF - 1):
            startg(0, j, j)

        def outer(t, carry):
            g0 = OUTER * t
            for idx in range(OUTER * GB):
                j = g0 * GB + idx
                gg = idx // GB
                row = idx % GB
                b = idx % NBUF
                if row == 0:
                    g = g0 + gg

                    @pl.when(g + 1 < G)
                    def _():
                        stage_wait((gg + 1) % 2)

                    @pl.when(g + 2 < G)
                    def _():
                        stage(g + 2, (gg + 2) % NRING, gg % 2)
                waitg(b)
                starts(gg, row, b)
                pidx = (idx + OUTER * GB - 1) % (OUTER * GB)
                nidx = idx + NBUF - 1
                nring = ((nidx // GB) % NRING)

                @pl.when((j >= 1) & (j + NBUF - 1 < NCHUNK))
                def _():
                    waits(pidx % NBUF)

                @pl.when(j + NBUF - 1 < NCHUNK)
                def _():
                    startg(nring, nidx % GB, pidx % NBUF)
            return carry

        lax.fori_loop(0, G // OUTER, outer, 0)
        for b in range(NBUF):
            waits((NCHUNK - NBUF + b) % NBUF)

    @pl.when(c == 0)
    def _():
        run(p_lo)

    @pl.when(c == 1)
    def _():
        run(p_hi)

    plsc.subcore_barrier()

    def writeout(out_hbm):
        for k in range(WPT):
            cidx = s + k * NS

            @pl.when(cidx < NWCH)
            def _():
                pltpu.sync_copy(acc.at[pl.ds(cidx * WCH, WCH)],
                                out_hbm.at[pl.ds(cidx * WCH, WCH)])

    @pl.when(c == 0)
    def _():
        writeout(out_lo)

    @pl.when(c == 1)
    def _():
        writeout(out_hi)


@functools.cache
def _sc_ag_gather_call():
    return functools.partial(
        pl.kernel,
        out_type=[
            jax.ShapeDtypeStruct((NAG, H), jnp.float32),
            jax.ShapeDtypeStruct((NAG, H), jnp.float32),
            jax.ShapeDtypeStruct((NAG, H), jnp.float32),
            jax.ShapeDtypeStruct((NAG, H), jnp.float32),
        ],
        mesh=_sc_mesh(),
        scratch_types=[
            pltpu.VMEM((AGT,), jnp.int32),
            pltpu.VMEM((AGT, H), jnp.float32),
            pltpu.VMEM((AGT, H), jnp.float32),
            pltpu.SemaphoreType.DMA,
        ],
    )(_sc_ag_gather_body)


def _sc_ag_gather_body(m_lo, m_hi, s_lo, s_hi, ag,
                       g_mlo, g_mhi, g_slo, g_shi,
                       agv, mbuf, sbuf, sem):
    c = lax.axis_index("c")
    s = lax.axis_index("s")
    base = s * AGT
    pltpu.sync_copy(ag.at[pl.ds(base, AGT)], agv)

    def run(m_hbm, s_hbm, gm_out, gs_out):
        pltpu.async_copy(m_hbm.at[agv], mbuf, sem).wait()
        pltpu.sync_copy(mbuf, gm_out.at[pl.ds(base, AGT)])
        pltpu.async_copy(s_hbm.at[agv], sbuf, sem).wait()
        pltpu.sync_copy(sbuf, gs_out.at[pl.ds(base, AGT)])

    @pl.when(c == 0)
    def _():
        run(m_lo, s_lo, g_mlo, g_slo)

    @pl.when(c == 1)
    def _():
        run(m_hi, s_hi, g_mhi, g_shi)


_R = 1000


def _tc_layer1(x, Wn, Ws, b):
    def body(x_ref, wn_ref, ws_ref, b_ref, plo_ref, phi_ref, s_ref):
        xb = x_ref[...]
        p = jnp.dot(xb, wn_ref[...], preferred_element_type=jnp.float32)
        plo_ref[...] = p[:, :H]
        phi_ref[...] = p[:, H:]
        s_ref[...] = jnp.dot(xb, ws_ref[...],
                             preferred_element_type=jnp.float32) + b_ref[...]

    f32 = jnp.float32
    return pl.pallas_call(
        body,
        grid=(N // _R,),
        in_specs=[
            pl.BlockSpec((_R, D), lambda i: (i, 0)),
            pl.BlockSpec((D, D), lambda i: (0, 0)),
            pl.BlockSpec((D, D), lambda i: (0, 0)),
            pl.BlockSpec((1, D), lambda i: (0, 0)),
        ],
        out_specs=[
            pl.BlockSpec((_R, H), lambda i: (i, 0)),
            pl.BlockSpec((_R, H), lambda i: (i, 0)),
            pl.BlockSpec((_R, D), lambda i: (i, 0)),
        ],
        out_shape=[
            jax.ShapeDtypeStruct((N, H), f32),
            jax.ShapeDtypeStruct((N, H), f32),
            jax.ShapeDtypeStruct((N, D), f32),
        ],
    )(x, Wn, Ws, b.reshape(1, D))


def _tc_layer2(m_lo, m_hi, s1, Wn, Ws, b):
    def body(mlo_ref, mhi_ref, s1_ref, wn_ref, ws_ref, b_ref,
             plo_ref, phi_ref, slo_ref, shi_ref):
        m = jnp.concatenate([mlo_ref[...], mhi_ref[...]], axis=1)
        h = jnp.maximum(m + s1_ref[...], 0.0)
        p = jnp.dot(h, wn_ref[...], preferred_element_type=jnp.float32)
        plo_ref[...] = p[:, :H]
        phi_ref[...] = p[:, H:]
        s2 = jnp.dot(h, ws_ref[...],
                     preferred_element_type=jnp.float32) + b_ref[...]
        slo_ref[...] = s2[:, :H]
        shi_ref[...] = s2[:, H:]

    f32 = jnp.float32
    half = pl.BlockSpec((_R, H), lambda i: (i, 0))
    return pl.pallas_call(
        body,
        grid=(N // _R,),
        in_specs=[
            half, half,
            pl.BlockSpec((_R, D), lambda i: (i, 0)),
            pl.BlockSpec((D, D), lambda i: (0, 0)),
            pl.BlockSpec((D, D), lambda i: (0, 0)),
            pl.BlockSpec((1, D), lambda i: (0, 0)),
        ],
        out_specs=[half, half, half, half],
        out_shape=[jax.ShapeDtypeStruct((N, H), f32)] * 4,
    )(m_lo, m_hi, s1, Wn, Ws, b.reshape(1, D))


def _tc_qhead(g_mlo, g_mhi, g_slo, g_shi, Wq1, bq1, Wq2, bq2):
    def body(mlo_ref, mhi_ref, slo_ref, shi_ref, w1_ref, b1_ref,
             w2_ref, b2_ref, q_ref):
        m = jnp.concatenate([mlo_ref[...], mhi_ref[...]], axis=1)
        sv = jnp.concatenate([slo_ref[...], shi_ref[...]], axis=1)
        agh = jnp.maximum(m + sv, 0.0)
        q1 = jnp.maximum(
            jnp.dot(agh, w1_ref[...], preferred_element_type=jnp.float32)
            + b1_ref[...], 0.0)
        q_ref[...] = jnp.dot(q1, w2_ref[...],
                             preferred_element_type=jnp.float32) + b2_ref[...]

    f32 = jnp.float32
    return pl.pallas_call(
        body,
        grid=(1,),
        in_specs=[
            pl.BlockSpec((NAG, H), lambda i: (0, 0)),
            pl.BlockSpec((NAG, H), lambda i: (0, 0)),
            pl.BlockSpec((NAG, H), lambda i: (0, 0)),
            pl.BlockSpec((NAG, H), lambda i: (0, 0)),
            pl.BlockSpec((D, H), lambda i: (0, 0)),
            pl.BlockSpec((1, H), lambda i: (0, 0)),
            pl.BlockSpec((H, 128), lambda i: (0, 0)),
            pl.BlockSpec((1, 128), lambda i: (0, 0)),
        ],
        out_specs=pl.BlockSpec((NAG, 128), lambda i: (0, 0)),
        out_shape=jax.ShapeDtypeStruct((NAG, 128), f32),
    )(g_mlo, g_mhi, g_slo, g_shi, Wq1, bq1.reshape(1, H), Wq2, bq2)


def kernel(x, edge_index, ag_nodes, W_nbr1, W_self1, b1,
           W_nbr2, W_self2, b2, Wq1, bq1, Wq2, bq2):
    pad = EPAD - E
    src2 = jnp.concatenate(
        [edge_index[0], jnp.zeros((pad,), edge_index.dtype)]
    ).reshape(NS * NCHUNK, CH)
    dst2 = jnp.concatenate(
        [edge_index[1], jnp.full((pad,), N, edge_index.dtype)]
    ).reshape(NS * NCHUNK, CH)
    ag_pad = jnp.concatenate(
        [ag_nodes, jnp.zeros((NAG - ag_nodes.shape[0],), ag_nodes.dtype)])

    p1_lo, p1_hi, s1 = _tc_layer1(x, W_nbr1, W_self1, b1)
    m1_lo, m1_hi = _sc_segsum_call()(p1_lo, p1_hi, src2, dst2)

    p2_lo, p2_hi, s2_lo, s2_hi = _tc_layer2(m1_lo, m1_hi, s1, W_nbr2, W_self2, b2)
    m2_lo, m2_hi = _sc_segsum_call()(p2_lo, p2_hi, src2, dst2)

    g_mlo, g_mhi, g_slo, g_shi = _sc_ag_gather_call()(
        m2_lo, m2_hi, s2_lo, s2_hi, ag_pad)
    Wq2_pad = jnp.zeros((H, 128), jnp.float32).at[:, :4].set(Wq2)
    bq2_pad = jnp.zeros((1, 128), jnp.float32).at[0, :4].set(bq2)
    q_full = _tc_qhead(g_mlo, g_mhi, g_slo, g_shi, Wq1, bq1, Wq2_pad, bq2_pad)
    return q_full[:ag_nodes.shape[0], :4]


# speedup vs baseline: 1.6367x; 1.6367x over previous
import functools

import jax
import jax.numpy as jnp
from jax import lax
from jax.experimental import pallas as pl
from jax.experimental.pallas import tpu as pltpu
from jax.experimental.pallas import tpu_sc as plsc

NC = 2
NS = 16
L = 16

N = 10000
E = 160000
D = 256
H = 128
NAG = 1024

CH = 105
NCHUNK = 96
EPAD = NS * NCHUNK * CH
NP = N + 8
GB = 8
G = NCHUNK // GB
NRING = 4
OUTER = 4
NBUF = 2
WCH = 40
NWCH = N // WCH
WPT = (NWCH + NS - 1) // NS
AGT = NAG // NS


def _sc_mesh():
    return plsc.VectorSubcoreMesh(core_axis_name="c", subcore_axis_name="s",
                                  num_cores=NC, num_subcores=NS)


@functools.cache
def _sc_segsum_call():
    return functools.partial(
        pl.kernel,
        out_type=[
            jax.ShapeDtypeStruct((N, H), jnp.float32),
            jax.ShapeDtypeStruct((N, H), jnp.float32),
        ],
        mesh=_sc_mesh(),
        scratch_types=[
            pltpu.VMEM((NCHUNK, CH), jnp.int32),
            pltpu.VMEM((GB, CH), jnp.int32),
            pltpu.VMEM((GB, CH), jnp.int32),
            pltpu.VMEM((GB, CH), jnp.int32),
            pltpu.VMEM((GB, CH), jnp.int32),
            pltpu.VMEM((CH, H), jnp.float32),
            pltpu.VMEM((CH, H), jnp.float32),
            pltpu.VMEM_SHARED((NP, H), jnp.float32),
            pltpu.SemaphoreType.DMA,
            pltpu.SemaphoreType.DMA,
            pltpu.SemaphoreType.DMA,
        ],
    )(_sc_segsum_body)


def _sc_segsum_body(p_lo, p_hi, src2, dst2, out_lo, out_hi,
                    srcv, dr0, dr1, dr2, dr3, gb0, gb1, acc,
                    gsem, is0, is1):
    drings = (dr0, dr1, dr2, dr3)
    bufs = (gb0, gb1)
    isems = (is0, is1)
    c = lax.axis_index("c")
    s = lax.axis_index("s")

    def zrow(r, carry):
        for j in range(H // L):
            gb0[r, pl.ds(j * L, L)] = jnp.zeros((L,), jnp.float32)
        return carry

    lax.fori_loop(0, WCH, zrow, 0)
    zsrc = gb0.at[pl.ds(0, WCH)]
    for k in range(WPT):
        cidx = s + k * NS

        @pl.when(cidx < NWCH)
        def _():
            pltpu.sync_copy(zsrc, acc.at[pl.ds(cidx * WCH, WCH)])
    pltpu.sync_copy(src2.at[pl.ds(s * NCHUNK, NCHUNK)], srcv)
    plsc.subcore_barrier()

    def stage(g, r, parity):
        base = pl.multiple_of(s * NCHUNK + g * GB, GB)
        pltpu.async_copy(dst2.at[pl.ds(base, GB)], drings[r], isems[parity])

    def stage_wait(parity):
        pltpu.make_async_copy(dst2.at[pl.ds(0, GB)], drings[0],
                              isems[parity]).wait()

    def run(p_hbm):
        def startg(j, b):
            pltpu.async_copy(p_hbm.at[srcv.at[j]], bufs[b], gsem)

        def waitg(b):
            pltpu.make_async_copy(p_hbm.at[srcv.at[0]], bufs[b],
                                  gsem).wait()

        stage(0, 0, 0)
        stage(1, 1, 1)
        stage_wait(0)
        startg(0, 0)

        def outer(t, carry):
            g0 = OUTER * t
            for idx in range(OUTER * GB):
                j = g0 * GB + idx
                gg = idx // GB
                row = idx % GB
                b = idx % NBUF
                if row == 0:
                    g = g0 + gg

                    @pl.when(g + 1 < G)
                    def _():
                        stage_wait((gg + 1) % 2)

                    @pl.when(g + 2 < G)
                    def _():
                        stage(g + 2, (gg + 2) % NRING, gg % 2)

                @pl.when(j + 1 < NCHUNK)
                def _():
                    startg(j + 1, 1 - b)

                waitg(b)
                pltpu.sync_copy(bufs[b], acc.at[drings[gg].at[row]],
                                add=True)
            return carry

        lax.fori_loop(0, G // OUTER, outer, 0)

    @pl.when(c == 0)
    def _():
        run(p_lo)

    @pl.when(c == 1)
    def _():
        run(p_hi)

    plsc.subcore_barrier()

    def writeout(out_hbm):
        for k in range(WPT):
            cidx = s + k * NS

            @pl.when(cidx < NWCH)
            def _():
                pltpu.sync_copy(acc.at[pl.ds(cidx * WCH, WCH)],
                                out_hbm.at[pl.ds(cidx * WCH, WCH)])

    @pl.when(c == 0)
    def _():
        writeout(out_lo)

    @pl.when(c == 1)
    def _():
        writeout(out_hi)


@functools.cache
def _sc_ag_gather_call():
    return functools.partial(
        pl.kernel,
        out_type=[
            jax.ShapeDtypeStruct((NAG, H), jnp.float32),
            jax.ShapeDtypeStruct((NAG, H), jnp.float32),
            jax.ShapeDtypeStruct((NAG, H), jnp.float32),
            jax.ShapeDtypeStruct((NAG, H), jnp.float32),
        ],
        mesh=_sc_mesh(),
        scratch_types=[
            pltpu.VMEM((AGT,), jnp.int32),
            pltpu.VMEM((AGT, H), jnp.float32),
            pltpu.VMEM((AGT, H), jnp.float32),
            pltpu.SemaphoreType.DMA,
        ],
    )(_sc_ag_gather_body)


def _sc_ag_gather_body(m_lo, m_hi, s_lo, s_hi, ag,
                       g_mlo, g_mhi, g_slo, g_shi,
                       agv, mbuf, sbuf, sem):
    c = lax.axis_index("c")
    s = lax.axis_index("s")
    base = s * AGT
    pltpu.sync_copy(ag.at[pl.ds(base, AGT)], agv)

    def run(m_hbm, s_hbm, gm_out, gs_out):
        pltpu.async_copy(m_hbm.at[agv], mbuf, sem).wait()
        pltpu.sync_copy(mbuf, gm_out.at[pl.ds(base, AGT)])
        pltpu.async_copy(s_hbm.at[agv], sbuf, sem).wait()
        pltpu.sync_copy(sbuf, gs_out.at[pl.ds(base, AGT)])

    @pl.when(c == 0)
    def _():
        run(m_lo, s_lo, g_mlo, g_slo)

    @pl.when(c == 1)
    def _():
        run(m_hi, s_hi, g_mhi, g_shi)


_R = 1000


def _tc_layer1(x, Wn, Ws, b):
    def body(x_ref, wn_ref, ws_ref, b_ref, plo_ref, phi_ref, s_ref):
        xb = x_ref[...]
        p = jnp.dot(xb, wn_ref[...], preferred_element_type=jnp.float32)
        plo_ref[...] = p[:, :H]
        phi_ref[...] = p[:, H:]
        s_ref[...] = jnp.dot(xb, ws_ref[...],
                             preferred_element_type=jnp.float32) + b_ref[...]

    f32 = jnp.float32
    return pl.pallas_call(
        body,
        grid=(N // _R,),
        in_specs=[
            pl.BlockSpec((_R, D), lambda i: (i, 0)),
            pl.BlockSpec((D, D), lambda i: (0, 0)),
            pl.BlockSpec((D, D), lambda i: (0, 0)),
            pl.BlockSpec((1, D), lambda i: (0, 0)),
        ],
        out_specs=[
            pl.BlockSpec((_R, H), lambda i: (i, 0)),
            pl.BlockSpec((_R, H), lambda i: (i, 0)),
            pl.BlockSpec((_R, D), lambda i: (i, 0)),
        ],
        out_shape=[
            jax.ShapeDtypeStruct((N, H), f32),
            jax.ShapeDtypeStruct((N, H), f32),
            jax.ShapeDtypeStruct((N, D), f32),
        ],
    )(x, Wn, Ws, b.reshape(1, D))


def _tc_layer2(m_lo, m_hi, s1, Wn, Ws, b):
    def body(mlo_ref, mhi_ref, s1_ref, wn_ref, ws_ref, b_ref,
             plo_ref, phi_ref, slo_ref, shi_ref):
        m = jnp.concatenate([mlo_ref[...], mhi_ref[...]], axis=1)
        h = jnp.maximum(m + s1_ref[...], 0.0)
        p = jnp.dot(h, wn_ref[...], preferred_element_type=jnp.float32)
        plo_ref[...] = p[:, :H]
        phi_ref[...] = p[:, H:]
        s2 = jnp.dot(h, ws_ref[...],
                     preferred_element_type=jnp.float32) + b_ref[...]
        slo_ref[...] = s2[:, :H]
        shi_ref[...] = s2[:, H:]

    f32 = jnp.float32
    half = pl.BlockSpec((_R, H), lambda i: (i, 0))
    return pl.pallas_call(
        body,
        grid=(N // _R,),
        in_specs=[
            half, half,
            pl.BlockSpec((_R, D), lambda i: (i, 0)),
            pl.BlockSpec((D, D), lambda i: (0, 0)),
            pl.BlockSpec((D, D), lambda i: (0, 0)),
            pl.BlockSpec((1, D), lambda i: (0, 0)),
        ],
        out_specs=[half, half, half, half],
        out_shape=[jax.ShapeDtypeStruct((N, H), f32)] * 4,
    )(m_lo, m_hi, s1, Wn, Ws, b.reshape(1, D))


def _tc_qhead(g_mlo, g_mhi, g_slo, g_shi, Wq1, bq1, Wq2, bq2):
    def body(mlo_ref, mhi_ref, slo_ref, shi_ref, w1_ref, b1_ref,
             w2_ref, b2_ref, q_ref):
        m = jnp.concatenate([mlo_ref[...], mhi_ref[...]], axis=1)
        sv = jnp.concatenate([slo_ref[...], shi_ref[...]], axis=1)
        agh = jnp.maximum(m + sv, 0.0)
        q1 = jnp.maximum(
            jnp.dot(agh, w1_ref[...], preferred_element_type=jnp.float32)
            + b1_ref[...], 0.0)
        q_ref[...] = jnp.dot(q1, w2_ref[...],
                             preferred_element_type=jnp.float32) + b2_ref[...]

    f32 = jnp.float32
    return pl.pallas_call(
        body,
        grid=(1,),
        in_specs=[
            pl.BlockSpec((NAG, H), lambda i: (0, 0)),
            pl.BlockSpec((NAG, H), lambda i: (0, 0)),
            pl.BlockSpec((NAG, H), lambda i: (0, 0)),
            pl.BlockSpec((NAG, H), lambda i: (0, 0)),
            pl.BlockSpec((D, H), lambda i: (0, 0)),
            pl.BlockSpec((1, H), lambda i: (0, 0)),
            pl.BlockSpec((H, 128), lambda i: (0, 0)),
            pl.BlockSpec((1, 128), lambda i: (0, 0)),
        ],
        out_specs=pl.BlockSpec((NAG, 128), lambda i: (0, 0)),
        out_shape=jax.ShapeDtypeStruct((NAG, 128), f32),
    )(g_mlo, g_mhi, g_slo, g_shi, Wq1, bq1.reshape(1, H), Wq2, bq2)


def kernel(x, edge_index, ag_nodes, W_nbr1, W_self1, b1,
           W_nbr2, W_self2, b2, Wq1, bq1, Wq2, bq2):
    pad = EPAD - E
    src2 = jnp.concatenate(
        [edge_index[0], jnp.zeros((pad,), edge_index.dtype)]
    ).reshape(NS * NCHUNK, CH)
    dst2 = jnp.concatenate(
        [edge_index[1], jnp.full((pad,), N, edge_index.dtype)]
    ).reshape(NS * NCHUNK, CH)
    ag_pad = jnp.concatenate(
        [ag_nodes, jnp.zeros((NAG - ag_nodes.shape[0],), ag_nodes.dtype)])

    p1_lo, p1_hi, s1 = _tc_layer1(x, W_nbr1, W_self1, b1)
    m1_lo, m1_hi = _sc_segsum_call()(p1_lo, p1_hi, src2, dst2)

    p2_lo, p2_hi, s2_lo, s2_hi = _tc_layer2(m1_lo, m1_hi, s1, W_nbr2, W_self2, b2)
    m2_lo, m2_hi = _sc_segsum_call()(p2_lo, p2_hi, src2, dst2)

    g_mlo, g_mhi, g_slo, g_shi = _sc_ag_gather_call()(
        m2_lo, m2_hi, s2_lo, s2_hi, ag_pad)
    Wq2_pad = jnp.zeros((H, 128), jnp.float32).at[:, :4].set(Wq2)
    bq2_pad = jnp.zeros((1, 128), jnp.float32).at[0, :4].set(bq2)
    q_full = _tc_qhead(g_mlo, g_mhi, g_slo, g_shi, Wq1, bq1, Wq2_pad, bq2_pad)
    return q_full[:ag_nodes.shape[0], :4]


# speedup vs baseline: 1.6434x; 1.0041x over previous
import functools

import jax
import jax.numpy as jnp
from jax import lax
from jax.experimental import pallas as pl
from jax.experimental.pallas import tpu as pltpu
from jax.experimental.pallas import tpu_sc as plsc

NC = 2
NS = 16
L = 16

N = 10000
E = 160000
D = 256
H = 128
HW = H // 2
NAG = 1024

CH = 105
NCHUNK = 96
EPAD = NS * NCHUNK * CH
NP = N + 8
GB = 8
G = NCHUNK // GB
NRING = 4
OUTER = 4
WCH = 40
NWCH = N // WCH
WPT = (NWCH + NS - 1) // NS
AGT = NAG // NS


def _sc_mesh():
    return plsc.VectorSubcoreMesh(core_axis_name="c", subcore_axis_name="s",
                                  num_cores=NC, num_subcores=NS)


@functools.cache
def _sc_segsum_call():
    return functools.partial(
        pl.kernel,
        out_type=[
            jax.ShapeDtypeStruct((N, H), jnp.float32),
            jax.ShapeDtypeStruct((N, H), jnp.float32),
        ],
        mesh=_sc_mesh(),
        scratch_types=[
            pltpu.VMEM((NCHUNK, CH), jnp.int32),
            pltpu.VMEM((GB, CH), jnp.int32),
            pltpu.VMEM((GB, CH), jnp.int32),
            pltpu.VMEM((GB, CH), jnp.int32),
            pltpu.VMEM((GB, CH), jnp.int32),
            pltpu.VMEM((CH, H), jnp.float32),
            pltpu.VMEM((CH, H), jnp.float32),
            pltpu.VMEM_SHARED((NP, H), jnp.float32),
            pltpu.SemaphoreType.DMA,
            pltpu.SemaphoreType.DMA,
            pltpu.SemaphoreType.DMA,
        ],
    )(_sc_segsum_body)


def _sc_segsum_body(p_lo, p_hi, src2, dst2, out_lo, out_hi,
                    srcv, dr0, dr1, dr2, dr3, gb0, gb1, acc,
                    gsem, is0, is1):
    drings = (dr0, dr1, dr2, dr3)
    bufs = (gb0, gb1)
    isems = (is0, is1)
    c = lax.axis_index("c")
    s = lax.axis_index("s")

    def zrow(r, carry):
        for j in range(H // L):
            gb0[r, pl.ds(j * L, L)] = jnp.zeros((L,), jnp.float32)
        return carry

    lax.fori_loop(0, WCH, zrow, 0)
    zsrc = gb0.at[pl.ds(0, WCH)]
    for k in range(WPT):
        cidx = s + k * NS

        @pl.when(cidx < NWCH)
        def _():
            pltpu.sync_copy(zsrc, acc.at[pl.ds(cidx * WCH, WCH)])
    pltpu.sync_copy(src2.at[pl.ds(s * NCHUNK, NCHUNK)], srcv)
    plsc.subcore_barrier()

    def stage(g, r, parity):
        base = pl.multiple_of(s * NCHUNK + g * GB, GB)
        pltpu.async_copy(dst2.at[pl.ds(base, GB)], drings[r], isems[parity])

    def stage_wait(parity):
        pltpu.make_async_copy(dst2.at[pl.ds(0, GB)], drings[0],
                              isems[parity]).wait()

    def run(p_hbm):
        def startg(j, b):
            pltpu.async_copy(p_hbm.at[srcv.at[j]], bufs[b], gsem)

        def waitg(b):
            pltpu.make_async_copy(p_hbm.at[srcv.at[0]], bufs[b],
                                  gsem).wait()

        stage(0, 0, 0)
        stage(1, 1, 1)
        stage_wait(0)
        startg(0, 0)

        def outer(t, carry):
            g0 = OUTER * t
            for idx in range(OUTER * GB):
                j = g0 * GB + idx
                gg = idx // GB
                row = idx % GB
                b = idx % 2
                if row == 0:
                    g = g0 + gg

                    @pl.when(g + 1 < G)
                    def _():
                        stage_wait((gg + 1) % 2)

                    @pl.when(g + 2 < G)
                    def _():
                        stage(g + 2, (gg + 2) % NRING, gg % 2)

                @pl.when(j + 1 < NCHUNK)
                def _():
                    startg(j + 1, 1 - b)

                waitg(b)
                pltpu.sync_copy(bufs[b], acc.at[drings[gg].at[row]],
                                add=True)
            return carry

        lax.fori_loop(0, G // OUTER, outer, 0)

    @pl.when(c == 0)
    def _():
        run(p_lo)

    @pl.when(c == 1)
    def _():
        run(p_hi)

    plsc.subcore_barrier()

    def writeout(out_hbm):
        for k in range(WPT):
            cidx = s + k * NS

            @pl.when(cidx < NWCH)
            def _():
                pltpu.sync_copy(acc.at[pl.ds(cidx * WCH, WCH)],
                                out_hbm.at[pl.ds(cidx * WCH, WCH)])

    @pl.when(c == 0)
    def _():
        writeout(out_lo)

    @pl.when(c == 1)
    def _():
        writeout(out_hi)


@functools.cache
def _sc_ag_gather_call():
    return functools.partial(
        pl.kernel,
        out_type=[
            jax.ShapeDtypeStruct((NAG, H), jnp.float32),
            jax.ShapeDtypeStruct((NAG, H), jnp.float32),
            jax.ShapeDtypeStruct((NAG, D), jnp.float32),
        ],
        mesh=_sc_mesh(),
        scratch_types=[
            pltpu.VMEM((AGT,), jnp.int32),
            pltpu.VMEM((AGT, H), jnp.float32),
            pltpu.VMEM((AGT, H), jnp.float32),
            pltpu.VMEM((AGT, D), jnp.float32),
            pltpu.SemaphoreType.DMA,
        ],
    )(_sc_ag_gather_body)


def _sc_ag_gather_body(m2_lo, m2_hi, s2, ag, g_mlo, g_mhi, g_s,
                       agv, lbuf, hbuf, sbuf, sem):
    c = lax.axis_index("c")
    s = lax.axis_index("s")
    base = s * AGT
    pltpu.sync_copy(ag.at[pl.ds(base, AGT)], agv)

    @pl.when(c == 0)
    def _():
        pltpu.async_copy(m2_lo.at[agv], lbuf, sem).wait()
        pltpu.sync_copy(lbuf, g_mlo.at[pl.ds(base, AGT)])
        pltpu.async_copy(m2_hi.at[agv], hbuf, sem).wait()
        pltpu.sync_copy(hbuf, g_mhi.at[pl.ds(base, AGT)])

    @pl.when(c == 1)
    def _():
        pltpu.async_copy(s2.at[agv], sbuf, sem).wait()
        pltpu.sync_copy(sbuf, g_s.at[pl.ds(base, AGT)])


_R = 1000


def _tc_layer1(x, Wn, Ws, b):
    def body(x_ref, wn_ref, ws_ref, b_ref, plo_ref, phi_ref, s_ref):
        xb = x_ref[...]
        p = jnp.dot(xb, wn_ref[...], preferred_element_type=jnp.float32)
        plo_ref[...] = p[:, :H]
        phi_ref[...] = p[:, H:]
        s_ref[...] = jnp.dot(xb, ws_ref[...],
                             preferred_element_type=jnp.float32) + b_ref[...]

    full = pl.BlockSpec((_R, D), lambda i: (i, 0))
    half = pl.BlockSpec((_R, H), lambda i: (i, 0))
    return pl.pallas_call(
        body,
        grid=(N // _R,),
        in_specs=[
            full,
            pl.BlockSpec((D, D), lambda i: (0, 0)),
            pl.BlockSpec((D, D), lambda i: (0, 0)),
            pl.BlockSpec((1, D), lambda i: (0, 0)),
        ],
        out_specs=[half, half, full],
        out_shape=[
            jax.ShapeDtypeStruct((N, H), jnp.float32),
            jax.ShapeDtypeStruct((N, H), jnp.float32),
            jax.ShapeDtypeStruct((N, D), jnp.float32),
        ],
    )(x, Wn, Ws, b.reshape(1, D))


def _tc_layer2(m1_lo, m1_hi, s1, Wn, Ws, b):
    def body(mlo_ref, mhi_ref, s1_ref, wn_ref, ws_ref, b_ref,
             plo_ref, phi_ref, s_ref):
        m = jnp.concatenate([mlo_ref[...], mhi_ref[...]], axis=1)
        h = jnp.maximum(m + s1_ref[...], 0.0)
        p = jnp.dot(h, wn_ref[...], preferred_element_type=jnp.float32)
        plo_ref[...] = p[:, :H]
        phi_ref[...] = p[:, H:]
        s_ref[...] = jnp.dot(h, ws_ref[...],
                             preferred_element_type=jnp.float32) + b_ref[...]

    full = pl.BlockSpec((_R, D), lambda i: (i, 0))
    half = pl.BlockSpec((_R, H), lambda i: (i, 0))
    return pl.pallas_call(
        body,
        grid=(N // _R,),
        in_specs=[
            half, half, full,
            pl.BlockSpec((D, D), lambda i: (0, 0)),
            pl.BlockSpec((D, D), lambda i: (0, 0)),
            pl.BlockSpec((1, D), lambda i: (0, 0)),
        ],
        out_specs=[half, half, full],
        out_shape=[
            jax.ShapeDtypeStruct((N, H), jnp.float32),
            jax.ShapeDtypeStruct((N, H), jnp.float32),
            jax.ShapeDtypeStruct((N, D), jnp.float32),
        ],
    )(m1_lo, m1_hi, s1, Wn, Ws, b.reshape(1, D))


def _tc_qhead(g_mlo, g_mhi, g_s, Wq1, bq1, Wq2, bq2):
    def body(mlo_ref, mhi_ref, s_ref, w1_ref, b1_ref, w2_ref, b2_ref,
             q_ref):
        m = jnp.concatenate([mlo_ref[...], mhi_ref[...]], axis=1)
        agh = jnp.maximum(m + s_ref[...], 0.0)
        q1 = jnp.maximum(
            jnp.dot(agh, w1_ref[...], preferred_element_type=jnp.float32)
            + b1_ref[...], 0.0)
        q_ref[...] = jnp.dot(q1, w2_ref[...],
                             preferred_element_type=jnp.float32) + b2_ref[...]

    return pl.pallas_call(
        body,
        grid=(1,),
        in_specs=[
            pl.BlockSpec((NAG, H), lambda i: (0, 0)),
            pl.BlockSpec((NAG, H), lambda i: (0, 0)),
            pl.BlockSpec((NAG, D), lambda i: (0, 0)),
            pl.BlockSpec((D, H), lambda i: (0, 0)),
            pl.BlockSpec((1, H), lambda i: (0, 0)),
            pl.BlockSpec((H, 128), lambda i: (0, 0)),
            pl.BlockSpec((1, 128), lambda i: (0, 0)),
        ],
        out_specs=pl.BlockSpec((NAG, 128), lambda i: (0, 0)),
        out_shape=jax.ShapeDtypeStruct((NAG, 128), jnp.float32),
    )(g_mlo, g_mhi, g_s, Wq1, bq1.reshape(1, H), Wq2, bq2)


def kernel(x, edge_index, ag_nodes, W_nbr1, W_self1, b1,
           W_nbr2, W_self2, b2, Wq1, bq1, Wq2, bq2):
    pad = EPAD - E
    src2 = jnp.concatenate(
        [edge_index[0], jnp.zeros((pad,), edge_index.dtype)]
    ).reshape(NS * NCHUNK, CH)
    dst2 = jnp.concatenate(
        [edge_index[1], jnp.full((pad,), N, edge_index.dtype)]
    ).reshape(NS * NCHUNK, CH)
    ag_pad = jnp.concatenate(
        [ag_nodes, jnp.zeros((NAG - ag_nodes.shape[0],), ag_nodes.dtype)])

    p1_lo, p1_hi, s1 = _tc_layer1(x, W_nbr1, W_self1, b1)
    m1_lo, m1_hi = _sc_segsum_call()(p1_lo, p1_hi, src2, dst2)

    p2_lo, p2_hi, s2 = _tc_layer2(m1_lo, m1_hi, s1, W_nbr2, W_self2, b2)
    m2_lo, m2_hi = _sc_segsum_call()(p2_lo, p2_hi, src2, dst2)

    g_mlo, g_mhi, g_s = _sc_ag_gather_call()(m2_lo, m2_hi, s2, ag_pad)
    Wq2_pad = jnp.zeros((H, 128), jnp.float32).at[:, :4].set(Wq2)
    bq2_pad = jnp.zeros((1, 128), jnp.float32).at[0, :4].set(bq2)
    q_full = _tc_qhead(g_mlo, g_mhi, g_s, Wq1, bq1, Wq2_pad, bq2_pad)
    return q_full[:ag_nodes.shape[0], :4]


# speedup vs baseline: 1.6439x; 1.0003x over previous
import functools

import jax
import jax.numpy as jnp
from jax import lax
from jax.experimental import pallas as pl
from jax.experimental.pallas import tpu as pltpu
from jax.experimental.pallas import tpu_sc as plsc

NC = 2
NS = 16
L = 16

N = 10000
E = 160000
D = 256
H = 128
HW = H // 2
NAG = 1024

CH = 105
NCHUNK = 96
EPAD = NS * NCHUNK * CH
NP = N + 8
GB = 8
G = NCHUNK // GB
NRING = 4
OUTER = 4
WCH = 40
NWCH = N // WCH
WPT = (NWCH + NS - 1) // NS
AGT = NAG // NS


def _sc_mesh():
    return plsc.VectorSubcoreMesh(core_axis_name="c", subcore_axis_name="s",
                                  num_cores=NC, num_subcores=NS)


@functools.cache
def _sc_segsum_call():
    return functools.partial(
        pl.kernel,
        out_type=[
            jax.ShapeDtypeStruct((N, H), jnp.float32),
            jax.ShapeDtypeStruct((N, H), jnp.float32),
        ],
        mesh=_sc_mesh(),
        scratch_types=[
            pltpu.VMEM((NCHUNK, CH), jnp.int32),
            pltpu.VMEM((GB, CH), jnp.int32),
            pltpu.VMEM((GB, CH), jnp.int32),
            pltpu.VMEM((GB, CH), jnp.int32),
            pltpu.VMEM((GB, CH), jnp.int32),
            pltpu.VMEM((CH, H), jnp.float32),
            pltpu.VMEM((CH, H), jnp.float32),
            pltpu.VMEM_SHARED((NP, H), jnp.float32),
            pltpu.SemaphoreType.DMA,
            pltpu.SemaphoreType.DMA,
            pltpu.SemaphoreType.DMA,
        ],
    )(_sc_segsum_body)


def _sc_segsum_body(p_lo, p_hi, src2, dst2, out_lo, out_hi,
                    srcv, dr0, dr1, dr2, dr3, gb0, gb1, acc,
                    gsem, is0, is1):
    drings = (dr0, dr1, dr2, dr3)
    bufs = (gb0, gb1)
    isems = (is0, is1)
    c = lax.axis_index("c")
    s = lax.axis_index("s")

    def zrow(r, carry):
        for j in range(H // L):
            gb0[r, pl.ds(j * L, L)] = jnp.zeros((L,), jnp.float32)
        return carry

    lax.fori_loop(0, WCH, zrow, 0)
    zsrc = gb0.at[pl.ds(0, WCH)]
    for k in range(WPT):
        cidx = s + k * NS

        @pl.when(cidx < NWCH)
        def _():
            pltpu.sync_copy(zsrc, acc.at[pl.ds(cidx * WCH, WCH)])
    pltpu.sync_copy(src2.at[pl.ds(s * NCHUNK, NCHUNK)], srcv)
    plsc.subcore_barrier()

    def stage(g, r, parity):
        base = pl.multiple_of(s * NCHUNK + g * GB, GB)
        pltpu.async_copy(dst2.at[pl.ds(base, GB)], drings[r], isems[parity])

    def stage_wait(parity):
        pltpu.make_async_copy(dst2.at[pl.ds(0, GB)], drings[0],
                              isems[parity]).wait()

    def run(p_hbm):
        def startg(j, b):
            pltpu.async_copy(p_hbm.at[srcv.at[j]], bufs[b], gsem)

        def waitg(b):
            pltpu.make_async_copy(p_hbm.at[srcv.at[0]], bufs[b],
                                  gsem).wait()

        stage(0, 0, 0)
        stage(1, 1, 1)
        stage_wait(0)
        startg(0, 0)

        def outer(t, carry):
            g0 = OUTER * t
            for idx in range(OUTER * GB):
                j = g0 * GB + idx
                gg = idx // GB
                row = idx % GB
                b = idx % 2
                if row == 0:
                    g = g0 + gg

                    @pl.when(g + 1 < G)
                    def _():
                        stage_wait((gg + 1) % 2)

                    @pl.when(g + 2 < G)
                    def _():
                        stage(g + 2, (gg + 2) % NRING, gg % 2)

                @pl.when(j + 1 < NCHUNK)
                def _():
                    startg(j + 1, 1 - b)

                waitg(b)
                pltpu.sync_copy(bufs[b], acc.at[drings[gg].at[row]],
                                add=True)
            return carry

        lax.fori_loop(0, G // OUTER, outer, 0)

    @pl.when(c == 0)
    def _():
        run(p_lo)

    @pl.when(c == 1)
    def _():
        run(p_hi)

    plsc.subcore_barrier()

    def writeout(out_hbm):
        for k in range(WPT):
            cidx = s + k * NS

            @pl.when(cidx < NWCH)
            def _():
                pltpu.sync_copy(acc.at[pl.ds(cidx * WCH, WCH)],
                                out_hbm.at[pl.ds(cidx * WCH, WCH)])

    @pl.when(c == 0)
    def _():
        writeout(out_lo)

    @pl.when(c == 1)
    def _():
        writeout(out_hi)


@functools.cache
def _sc_ag_gather_call():
    return functools.partial(
        pl.kernel,
        out_type=[
            jax.ShapeDtypeStruct((NAG, H), jnp.float32),
            jax.ShapeDtypeStruct((NAG, H), jnp.float32),
            jax.ShapeDtypeStruct((NAG, D), jnp.float32),
        ],
        mesh=_sc_mesh(),
        scratch_types=[
            pltpu.VMEM((AGT,), jnp.int32),
            pltpu.VMEM((AGT, H), jnp.float32),
            pltpu.VMEM((AGT, H), jnp.float32),
            pltpu.VMEM((AGT, D), jnp.float32),
            pltpu.SemaphoreType.DMA,
        ],
    )(_sc_ag_gather_body)


def _sc_ag_gather_body(m2_lo, m2_hi, s2, ag, g_mlo, g_mhi, g_s,
                       agv, lbuf, hbuf, sbuf, sem):
    c = lax.axis_index("c")
    s = lax.axis_index("s")
    base = s * AGT
    pltpu.sync_copy(ag.at[pl.ds(base, AGT)], agv)

    @pl.when(c == 0)
    def _():
        pltpu.async_copy(m2_lo.at[agv], lbuf, sem).wait()
        pltpu.sync_copy(lbuf, g_mlo.at[pl.ds(base, AGT)])
        pltpu.async_copy(m2_hi.at[agv], hbuf, sem).wait()
        pltpu.sync_copy(hbuf, g_mhi.at[pl.ds(base, AGT)])

    @pl.when(c == 1)
    def _():
        pltpu.async_copy(s2.at[agv], sbuf, sem).wait()
        pltpu.sync_copy(sbuf, g_s.at[pl.ds(base, AGT)])


_R = 1000


def _tc_nbr1(x, Wn):
    def body(x_ref, wn_ref, plo_ref, phi_ref):
        p = jnp.dot(x_ref[...], wn_ref[...],
                    preferred_element_type=jnp.float32)
        plo_ref[...] = p[:, :H]
        phi_ref[...] = p[:, H:]

    full = pl.BlockSpec((_R, D), lambda i: (i, 0))
    half = pl.BlockSpec((_R, H), lambda i: (i, 0))
    return pl.pallas_call(
        body,
        grid=(N // _R,),
        in_specs=[full, pl.BlockSpec((D, D), lambda i: (0, 0))],
        out_specs=[half, half],
        out_shape=[
            jax.ShapeDtypeStruct((N, H), jnp.float32),
            jax.ShapeDtypeStruct((N, H), jnp.float32),
        ],
    )(x, Wn)


def _tc_self1(x, Ws, b):
    def body(x_ref, ws_ref, b_ref, s_ref):
        s_ref[...] = jnp.dot(x_ref[...], ws_ref[...],
                             preferred_element_type=jnp.float32) + b_ref[...]

    full = pl.BlockSpec((_R, D), lambda i: (i, 0))
    return pl.pallas_call(
        body,
        grid=(N // _R,),
        in_specs=[
            full,
            pl.BlockSpec((D, D), lambda i: (0, 0)),
            pl.BlockSpec((1, D), lambda i: (0, 0)),
        ],
        out_specs=full,
        out_shape=jax.ShapeDtypeStruct((N, D), jnp.float32),
    )(x, Ws, b.reshape(1, D))


def _tc_layer2(m1_lo, m1_hi, s1, Wn, Ws, b):
    def body(mlo_ref, mhi_ref, s1_ref, wn_ref, ws_ref, b_ref,
             plo_ref, phi_ref, s_ref):
        m = jnp.concatenate([mlo_ref[...], mhi_ref[...]], axis=1)
        h = jnp.maximum(m + s1_ref[...], 0.0)
        p = jnp.dot(h, wn_ref[...], preferred_element_type=jnp.float32)
        plo_ref[...] = p[:, :H]
        phi_ref[...] = p[:, H:]
        s_ref[...] = jnp.dot(h, ws_ref[...],
                             preferred_element_type=jnp.float32) + b_ref[...]

    full = pl.BlockSpec((_R, D), lambda i: (i, 0))
    half = pl.BlockSpec((_R, H), lambda i: (i, 0))
    return pl.pallas_call(
        body,
        grid=(N // _R,),
        in_specs=[
            half, half, full,
            pl.BlockSpec((D, D), lambda i: (0, 0)),
            pl.BlockSpec((D, D), lambda i: (0, 0)),
            pl.BlockSpec((1, D), lambda i: (0, 0)),
        ],
        out_specs=[half, half, full],
        out_shape=[
            jax.ShapeDtypeStruct((N, H), jnp.float32),
            jax.ShapeDtypeStruct((N, H), jnp.float32),
            jax.ShapeDtypeStruct((N, D), jnp.float32),
        ],
    )(m1_lo, m1_hi, s1, Wn, Ws, b.reshape(1, D))


def _tc_qhead(g_mlo, g_mhi, g_s, Wq1, bq1, Wq2, bq2):
    def body(mlo_ref, mhi_ref, s_ref, w1_ref, b1_ref, w2_ref, b2_ref,
             q_ref):
        m = jnp.concatenate([mlo_ref[...], mhi_ref[...]], axis=1)
        agh = jnp.maximum(m + s_ref[...], 0.0)
        q1 = jnp.maximum(
            jnp.dot(agh, w1_ref[...], preferred_element_type=jnp.float32)
            + b1_ref[...], 0.0)
        q_ref[...] = jnp.dot(q1, w2_ref[...],
                             preferred_element_type=jnp.float32) + b2_ref[...]

    return pl.pallas_call(
        body,
        grid=(1,),
        in_specs=[
            pl.BlockSpec((NAG, H), lambda i: (0, 0)),
            pl.BlockSpec((NAG, H), lambda i: (0, 0)),
            pl.BlockSpec((NAG, D), lambda i: (0, 0)),
            pl.BlockSpec((D, H), lambda i: (0, 0)),
            pl.BlockSpec((1, H), lambda i: (0, 0)),
            pl.BlockSpec((H, 128), lambda i: (0, 0)),
            pl.BlockSpec((1, 128), lambda i: (0, 0)),
        ],
        out_specs=pl.BlockSpec((NAG, 128), lambda i: (0, 0)),
        out_shape=jax.ShapeDtypeStruct((NAG, 128), jnp.float32),
    )(g_mlo, g_mhi, g_s, Wq1, bq1.reshape(1, H), Wq2, bq2)


def kernel(x, edge_index, ag_nodes, W_nbr1, W_self1, b1,
           W_nbr2, W_self2, b2, Wq1, bq1, Wq2, bq2):
    pad = EPAD - E
    src2 = jnp.concatenate(
        [edge_index[0], jnp.zeros((pad,), edge_index.dtype)]
    ).reshape(NS * NCHUNK, CH)
    dst2 = jnp.concatenate(
        [edge_index[1], jnp.full((pad,), N, edge_index.dtype)]
    ).reshape(NS * NCHUNK, CH)
    ag_pad = jnp.concatenate(
        [ag_nodes, jnp.zeros((NAG - ag_nodes.shape[0],), ag_nodes.dtype)])

    p1_lo, p1_hi = _tc_nbr1(x, W_nbr1)
    m1_lo, m1_hi = _sc_segsum_call()(p1_lo, p1_hi, src2, dst2)
    s1 = _tc_self1(x, W_self1, b1)

    p2_lo, p2_hi, s2 = _tc_layer2(m1_lo, m1_hi, s1, W_nbr2, W_self2, b2)
    m2_lo, m2_hi = _sc_segsum_call()(p2_lo, p2_hi, src2, dst2)

    g_mlo, g_mhi, g_s = _sc_ag_gather_call()(m2_lo, m2_hi, s2, ag_pad)
    Wq2_pad = jnp.zeros((H, 128), jnp.float32).at[:, :4].set(Wq2)
    bq2_pad = jnp.zeros((1, 128), jnp.float32).at[0, :4].set(bq2)
    q_full = _tc_qhead(g_mlo, g_mhi, g_s, Wq1, bq1, Wq2_pad, bq2_pad)
    return q_full[:ag_nodes.shape[0], :4]


# speedup vs baseline: 1.6449x; 1.0006x over previous
import functools

import jax
import jax.numpy as jnp
from jax import lax
from jax.experimental import pallas as pl
from jax.experimental.pallas import tpu as pltpu
from jax.experimental.pallas import tpu_sc as plsc

NC = 2
NS = 16
L = 16

N = 10000
E = 160000
D = 256
H = 128
NAG = 1024

CH = 105
NCHUNK = 96
EPAD = NS * NCHUNK * CH
NP = N + 8
GB = 8
G = NCHUNK // GB
NRING = 4
OUTER = 4
WCH = 40
NWCH = N // WCH
WPT = (NWCH + NS - 1) // NS
AGT = NAG // NS


def _sc_mesh():
    return plsc.VectorSubcoreMesh(core_axis_name="c", subcore_axis_name="s",
                                  num_cores=NC, num_subcores=NS)


@functools.cache
def _sc_segsum_call():
    return functools.partial(
        pl.kernel,
        out_type=[
            jax.ShapeDtypeStruct((N, H), jnp.float32),
            jax.ShapeDtypeStruct((N, H), jnp.float32),
        ],
        mesh=_sc_mesh(),
        scratch_types=[
            pltpu.VMEM((NCHUNK, CH), jnp.int32),
            pltpu.VMEM((GB, CH), jnp.int32),
            pltpu.VMEM((GB, CH), jnp.int32),
            pltpu.VMEM((GB, CH), jnp.int32),
            pltpu.VMEM((GB, CH), jnp.int32),
            pltpu.VMEM((CH, H), jnp.float32),
            pltpu.VMEM((CH, H), jnp.float32),
            pltpu.VMEM_SHARED((NP, H), jnp.float32),
            pltpu.SemaphoreType.DMA,
            pltpu.SemaphoreType.DMA,
            pltpu.SemaphoreType.DMA,
        ],
    )(_sc_segsum_body)


def _sc_segsum_body(p_lo, p_hi, src2, dst2, out_lo, out_hi,
                    srcv, dr0, dr1, dr2, dr3, gb0, gb1, acc,
                    gsem, is0, is1):
    drings = (dr0, dr1, dr2, dr3)
    bufs = (gb0, gb1)
    isems = (is0, is1)
    c = lax.axis_index("c")
    s = lax.axis_index("s")

    def zrow(r, carry):
        for j in range(H // L):
            gb0[r, pl.ds(j * L, L)] = jnp.zeros((L,), jnp.float32)
        return carry

    lax.fori_loop(0, WCH, zrow, 0)
    zsrc = gb0.at[pl.ds(0, WCH)]
    for k in range(WPT):
        cidx = s + k * NS

        @pl.when(cidx < NWCH)
        def _():
            pltpu.sync_copy(zsrc, acc.at[pl.ds(cidx * WCH, WCH)])
    pltpu.sync_copy(src2.at[pl.ds(s * NCHUNK, NCHUNK)], srcv)
    plsc.subcore_barrier()

    def stage(g, r, parity):
        base = pl.multiple_of(s * NCHUNK + g * GB, GB)
        pltpu.async_copy(dst2.at[pl.ds(base, GB)], drings[r], isems[parity])

    def stage_wait(parity):
        pltpu.make_async_copy(dst2.at[pl.ds(0, GB)], drings[0],
                              isems[parity]).wait()

    def run(p_hbm):
        def startg(j, b):
            pltpu.async_copy(p_hbm.at[srcv.at[j]], bufs[b], gsem)

        def waitg(b):
            pltpu.make_async_copy(p_hbm.at[srcv.at[0]], bufs[b],
                                  gsem).wait()

        stage(0, 0, 0)
        stage(1, 1, 1)
        stage_wait(0)
        startg(0, 0)

        def outer(t, carry):
            g0 = OUTER * t
            for idx in range(OUTER * GB):
                j = g0 * GB + idx
                gg = idx // GB
                row = idx % GB
                b = idx % 2
                if row == 0:
                    g = g0 + gg

                    @pl.when(g + 1 < G)
                    def _():
                        stage_wait((gg + 1) % 2)

                    @pl.when(g + 2 < G)
                    def _():
                        stage(g + 2, (gg + 2) % NRING, gg % 2)

                @pl.when(j + 1 < NCHUNK)
                def _():
                    startg(j + 1, 1 - b)

                waitg(b)
                pltpu.sync_copy(bufs[b], acc.at[drings[gg].at[row]],
                                add=True)
            return carry

        lax.fori_loop(0, G // OUTER, outer, 0)

    @pl.when(c == 0)
    def _():
        run(p_lo)

    @pl.when(c == 1)
    def _():
        run(p_hi)

    plsc.subcore_barrier()

    def writeout(out_hbm):
        for k in range(WPT):
            cidx = s + k * NS

            @pl.when(cidx < NWCH)
            def _():
                pltpu.sync_copy(acc.at[pl.ds(cidx * WCH, WCH)],
                                out_hbm.at[pl.ds(cidx * WCH, WCH)])

    @pl.when(c == 0)
    def _():
        writeout(out_lo)

    @pl.when(c == 1)
    def _():
        writeout(out_hi)


@functools.cache
def _sc_ag_gather_call():
    return functools.partial(
        pl.kernel,
        out_type=[
            jax.ShapeDtypeStruct((NAG, H), jnp.float32),
            jax.ShapeDtypeStruct((NAG, H), jnp.float32),
            jax.ShapeDtypeStruct((NAG, D), jnp.float32),
        ],
        mesh=_sc_mesh(),
        scratch_types=[
            pltpu.VMEM((AGT,), jnp.int32),
            pltpu.VMEM((AGT, H), jnp.float32),
            pltpu.VMEM((AGT, H), jnp.float32),
            pltpu.VMEM((AGT, D), jnp.float32),
            pltpu.SemaphoreType.DMA,
        ],
    )(_sc_ag_gather_body)


def _sc_ag_gather_body(m2_lo, m2_hi, s2, ag, g_mlo, g_mhi, g_s,
                       agv, lbuf, hbuf, sbuf, sem):
    c = lax.axis_index("c")
    s = lax.axis_index("s")
    base = s * AGT
    pltpu.sync_copy(ag.at[pl.ds(base, AGT)], agv)

    @pl.when(c == 0)
    def _():
        pltpu.async_copy(m2_lo.at[agv], lbuf, sem).wait()
        pltpu.sync_copy(lbuf, g_mlo.at[pl.ds(base, AGT)])
        pltpu.async_copy(m2_hi.at[agv], hbuf, sem).wait()
        pltpu.sync_copy(hbuf, g_mhi.at[pl.ds(base, AGT)])

    @pl.when(c == 1)
    def _():
        pltpu.async_copy(s2.at[agv], sbuf, sem).wait()
        pltpu.sync_copy(sbuf, g_s.at[pl.ds(base, AGT)])


_R = 1000


def _tc_nbr1(x, Wn):
    def body(x_ref, wn_ref, plo_ref, phi_ref):
        p = jnp.dot(x_ref[...], wn_ref[...],
                    preferred_element_type=jnp.float32)
        plo_ref[...] = p[:, :H]
        phi_ref[...] = p[:, H:]

    full = pl.BlockSpec((_R, D), lambda i: (i, 0))
    half = pl.BlockSpec((_R, H), lambda i: (i, 0))
    return pl.pallas_call(
        body,
        grid=(N // _R,),
        in_specs=[full, pl.BlockSpec((D, D), lambda i: (0, 0))],
        out_specs=[half, half],
        out_shape=[
            jax.ShapeDtypeStruct((N, H), jnp.float32),
            jax.ShapeDtypeStruct((N, H), jnp.float32),
        ],
    )(x, Wn)


def _tc_self1(x, Ws, b):
    def body(x_ref, ws_ref, b_ref, s_ref):
        s_ref[...] = jnp.dot(x_ref[...], ws_ref[...],
                             preferred_element_type=jnp.float32) + b_ref[...]

    full = pl.BlockSpec((_R, D), lambda i: (i, 0))
    return pl.pallas_call(
        body,
        grid=(N // _R,),
        in_specs=[
            full,
            pl.BlockSpec((D, D), lambda i: (0, 0)),
            pl.BlockSpec((1, D), lambda i: (0, 0)),
        ],
        out_specs=full,
        out_shape=jax.ShapeDtypeStruct((N, D), jnp.float32),
    )(x, Ws, b.reshape(1, D))


def _tc_layer2(m1_lo, m1_hi, s1, Wn, Ws, b):
    def body(mlo_ref, mhi_ref, s1_ref, wn_ref, ws_ref, b_ref,
             plo_ref, phi_ref, s_ref):
        m = jnp.concatenate([mlo_ref[...], mhi_ref[...]], axis=1)
        h = jnp.maximum(m + s1_ref[...], 0.0)
        p = jnp.dot(h, wn_ref[...], preferred_element_type=jnp.float32)
        plo_ref[...] = p[:, :H]
        phi_ref[...] = p[:, H:]
        s_ref[...] = jnp.dot(h, ws_ref[...],
                             preferred_element_type=jnp.float32) + b_ref[...]

    full = pl.BlockSpec((_R, D), lambda i: (i, 0))
    half = pl.BlockSpec((_R, H), lambda i: (i, 0))
    return pl.pallas_call(
        body,
        grid=(N // _R,),
        in_specs=[
            half, half, full,
            pl.BlockSpec((D, D), lambda i: (0, 0)),
            pl.BlockSpec((D, D), lambda i: (0, 0)),
            pl.BlockSpec((1, D), lambda i: (0, 0)),
        ],
        out_specs=[half, half, full],
        out_shape=[
            jax.ShapeDtypeStruct((N, H), jnp.float32),
            jax.ShapeDtypeStruct((N, H), jnp.float32),
            jax.ShapeDtypeStruct((N, D), jnp.float32),
        ],
    )(m1_lo, m1_hi, s1, Wn, Ws, b.reshape(1, D))


def _tc_qhead(g_mlo, g_mhi, g_s, Wq1, bq1, Wq2, bq2):
    def body(mlo_ref, mhi_ref, s_ref, w1_ref, b1_ref, w2_ref, b2_ref,
             q_ref):
        m = jnp.concatenate([mlo_ref[...], mhi_ref[...]], axis=1)
        agh = jnp.maximum(m + s_ref[...], 0.0)
        q1 = jnp.maximum(
            jnp.dot(agh, w1_ref[...], preferred_element_type=jnp.float32)
            + b1_ref[...], 0.0)
        q_ref[...] = jnp.dot(q1, w2_ref[...],
                             preferred_element_type=jnp.float32) + b2_ref[...]

    return pl.pallas_call(
        body,
        grid=(1,),
        in_specs=[
            pl.BlockSpec((NAG, H), lambda i: (0, 0)),
            pl.BlockSpec((NAG, H), lambda i: (0, 0)),
            pl.BlockSpec((NAG, D), lambda i: (0, 0)),
            pl.BlockSpec((D, H), lambda i: (0, 0)),
            pl.BlockSpec((1, H), lambda i: (0, 0)),
            pl.BlockSpec((H, 128), lambda i: (0, 0)),
            pl.BlockSpec((1, 128), lambda i: (0, 0)),
        ],
        out_specs=pl.BlockSpec((NAG, 128), lambda i: (0, 0)),
        out_shape=jax.ShapeDtypeStruct((NAG, 128), jnp.float32),
    )(g_mlo, g_mhi, g_s, Wq1, bq1.reshape(1, H), Wq2, bq2)


def kernel(x, edge_index, ag_nodes, W_nbr1, W_self1, b1,
           W_nbr2, W_self2, b2, Wq1, bq1, Wq2, bq2):
    pad = EPAD - E
    src2 = jnp.concatenate(
        [edge_index[0], jnp.zeros((pad,), edge_index.dtype)]
    ).reshape(NS * NCHUNK, CH)
    dst2 = jnp.concatenate(
        [edge_index[1], jnp.full((pad,), N, edge_index.dtype)]
    ).reshape(NS * NCHUNK, CH)
    ag_pad = jnp.concatenate(
        [ag_nodes, jnp.zeros((NAG - ag_nodes.shape[0],), ag_nodes.dtype)])

    p1_lo, p1_hi = _tc_nbr1(x, W_nbr1)
    m1_lo, m1_hi = _sc_segsum_call()(p1_lo, p1_hi, src2, dst2)
    s1 = _tc_self1(x, W_self1, b1)

    p2_lo, p2_hi, s2 = _tc_layer2(m1_lo, m1_hi, s1, W_nbr2, W_self2, b2)
    m2_lo, m2_hi = _sc_segsum_call()(p2_lo, p2_hi, src2, dst2)

    g_mlo, g_mhi, g_s = _sc_ag_gather_call()(m2_lo, m2_hi, s2, ag_pad)
    Wq2_pad = jnp.zeros((H, 128), jnp.float32).at[:, :4].set(Wq2)
    bq2_pad = jnp.zeros((1, 128), jnp.float32).at[0, :4].set(bq2)
    q_full = _tc_qhead(g_mlo, g_mhi, g_s, Wq1, bq1, Wq2_pad, bq2_pad)
    return q_full[:ag_nodes.shape[0], :4]
